# Initial kernel scaffold; baseline (speedup 1.0000x reference)
import jax
import jax.numpy as jnp
from jax.experimental import pallas as pl


def kernel(x):
    raise NotImplementedError("write your pallas kernel here")


# speedup vs baseline: 1.3493x; 1.3493x over previous
import jax
import jax.numpy as jnp
import numpy as np
from jax.experimental import pallas as pl
from jax.experimental.pallas import tpu as pltpu

_N, _C, _H, _W = 32, 256, 64, 64
_HW = _H * _W
_EPS = 1e-05

_PERM_NP = np.asarray(jax.random.permutation(jax.random.key(42), _N),
                      dtype=np.int32)


def _stats_body(x_ref, mean_ref, std_ref, rstd_ref):
    blk = x_ref[...]
    s = jnp.sum(blk, axis=0)
    sq = jnp.sum(blk * blk, axis=0)
    m = s * (1.0 / _C)
    var = (sq - _C * m * m) * (1.0 / (_C - 1))
    std = jnp.sqrt(var + _EPS)
    mean_ref[...] = m.reshape(1, _HW)
    std_ref[...] = std.reshape(1, _HW)
    rstd_ref[...] = (1.0 / std).reshape(1, _HW)


def _apply_body(perm_ref, x_ref, mean_ref, std_ref, rstd_ref, o_ref):
    i = pl.program_id(0)
    j = perm_ref[i]
    blk = x_ref[...]
    m_j = mean_ref[j, 0, :]
    r_j = rstd_ref[j, 0, :]
    m_i = mean_ref[i, 0, :]
    s_i = std_ref[i, 0, :]
    o_ref[...] = (blk - m_j) * (r_j * s_i) + m_i


def kernel(x):
    n, c, h, w = x.shape
    xr = x.reshape(n, c, h * w)

    stats_shape = jax.ShapeDtypeStruct((n, 1, _HW), jnp.float32)
    mean, std, rstd = pl.pallas_call(
        _stats_body,
        grid=(n,),
        in_specs=[pl.BlockSpec((None, c, _HW), lambda i: (i, 0, 0))],
        out_specs=[
            pl.BlockSpec((None, 1, _HW), lambda i: (i, 0, 0)),
            pl.BlockSpec((None, 1, _HW), lambda i: (i, 0, 0)),
            pl.BlockSpec((None, 1, _HW), lambda i: (i, 0, 0)),
        ],
        out_shape=[stats_shape, stats_shape, stats_shape],
    )(xr)

    perm = jnp.asarray(_PERM_NP)
    out = pl.pallas_call(
        _apply_body,
        grid_spec=pltpu.PrefetchScalarGridSpec(
            num_scalar_prefetch=1,
            grid=(n,),
            in_specs=[
                pl.BlockSpec((None, c, _HW), lambda i, p: (p[i], 0, 0)),
                pl.BlockSpec((n, 1, _HW), lambda i, p: (0, 0, 0)),
                pl.BlockSpec((n, 1, _HW), lambda i, p: (0, 0, 0)),
                pl.BlockSpec((n, 1, _HW), lambda i, p: (0, 0, 0)),
            ],
            out_specs=pl.BlockSpec((None, c, _HW), lambda i, p: (i, 0, 0)),
        ),
        out_shape=jax.ShapeDtypeStruct((n, c, h * w), jnp.float32),
    )(perm, xr, mean, std, rstd)

    return out.reshape(n, c, h, w)


# speedup vs baseline: 1.5439x; 1.1442x over previous
import jax
import jax.numpy as jnp
import numpy as np
from jax.experimental import pallas as pl
from jax.experimental.pallas import tpu as pltpu

_N, _C, _H, _W = 32, 256, 64, 64
_HW = _H * _W
_EPS = 1e-05

_PERM_NP = np.asarray(jax.random.permutation(jax.random.key(42), _N),
                      dtype=np.int32)


def _cycle_plan(perm):
    n = len(perm)
    visited = [False] * n
    load, out_idx, emit_normal, emit_first, save_first = [], [], [], [], []
    prev_cycle_last = None
    for s in range(n):
        if visited[s]:
            continue
        cyc = []
        a = s
        while not visited[a]:
            visited[a] = True
            cyc.append(a)
            a = int(perm[a])
        for m, a in enumerate(cyc):
            load.append(a)
            if m == 0:
                save_first.append(1)
                emit_normal.append(0)
                if prev_cycle_last is None:
                    emit_first.append(0)
                    out_idx.append(-1)
                else:
                    emit_first.append(1)
                    out_idx.append(prev_cycle_last)
            else:
                save_first.append(0)
                emit_first.append(0)
                emit_normal.append(1)
                out_idx.append(cyc[m - 1])
        prev_cycle_last = cyc[-1]
    load.append(load[-1])
    save_first.append(0)
    emit_normal.append(0)
    emit_first.append(1)
    out_idx.append(prev_cycle_last)
    out_idx[0] = out_idx[1]
    idx = np.asarray([load, out_idx], dtype=np.int32)
    flg = np.asarray([emit_normal, emit_first, save_first], dtype=np.int32)
    return idx, flg


_IDX_NP, _FLG_NP = _cycle_plan(_PERM_NP)
_STEPS = _IDX_NP.shape[1]


def _body(idx_ref, flg_ref, x_ref, o_ref,
          xfirst, first_m, first_r, prev_m, prev_s):
    t = pl.program_id(0)
    cur = x_ref[...]
    s = jnp.sum(cur, axis=0)
    sq = jnp.sum(cur * cur, axis=0)
    m_cur = (s * (1.0 / _C)).reshape(1, _HW)
    var = (sq.reshape(1, _HW) - _C * m_cur * m_cur) * (1.0 / (_C - 1))
    s_cur = jnp.sqrt(var + _EPS)
    r_cur = 1.0 / s_cur

    @pl.when(flg_ref[0, t] == 1)
    def _():
        o_ref[...] = (cur - m_cur) * (r_cur * prev_s[...]) + prev_m[...]

    @pl.when(flg_ref[1, t] == 1)
    def _():
        o_ref[...] = ((xfirst[...] - first_m[...])
                      * (first_r[...] * prev_s[...]) + prev_m[...])

    @pl.when(flg_ref[2, t] == 1)
    def _():
        xfirst[...] = cur
        first_m[...] = m_cur
        first_r[...] = r_cur

    prev_m[...] = m_cur
    prev_s[...] = s_cur


def kernel(x):
    n, c, h, w = x.shape
    xr = x.reshape(n, c, h * w)
    idx = jnp.asarray(_IDX_NP)
    flg = jnp.asarray(_FLG_NP)
    out = pl.pallas_call(
        _body,
        grid_spec=pltpu.PrefetchScalarGridSpec(
            num_scalar_prefetch=2,
            grid=(_STEPS,),
            in_specs=[
                pl.BlockSpec((None, c, _HW), lambda t, i, f: (i[0, t], 0, 0)),
            ],
            out_specs=pl.BlockSpec((None, c, _HW), lambda t, i, f: (i[1, t], 0, 0)),
            scratch_shapes=[
                pltpu.VMEM((c, _HW), jnp.float32),
                pltpu.VMEM((1, _HW), jnp.float32),
                pltpu.VMEM((1, _HW), jnp.float32),
                pltpu.VMEM((1, _HW), jnp.float32),
                pltpu.VMEM((1, _HW), jnp.float32),
            ],
        ),
        out_shape=jax.ShapeDtypeStruct((n, c, h * w), jnp.float32),
    )(idx, flg, xr)

    return out.reshape(n, c, h, w)
